# Initial kernel scaffold; baseline (speedup 1.0000x reference)
import jax
import jax.numpy as jnp
from jax.experimental import pallas as pl


def kernel(in_x, gemm_edges, W1_0, b1_0, W2_0, b2_0, W1_1, b1_1, W2_1, b2_1):
    raise NotImplementedError("write your pallas kernel here")


# speedup vs baseline: 3.5292x; 3.5292x over previous
import functools

import jax
import jax.numpy as jnp
from jax import lax
from jax.experimental import pallas as pl
from jax.experimental.pallas import tpu as pltpu
from jax.experimental.pallas import tpu_sc as plsc

LEAKY = 0.2
EPS = 1e-5
E_REAL = 10000
E_PAD = 10240
N_WORKERS = 32
E_PER_W = E_PAD // N_WORKERS
CHUNK = 64
N_CHUNKS = E_PER_W // CHUNK
E_TILE = 256


def _sc_gather(xT, idx4):
    C = xT.shape[1]
    mesh = plsc.VectorSubcoreMesh(core_axis_name="c", subcore_axis_name="s")

    @functools.partial(
        pl.kernel,
        mesh=mesh,
        out_type=[jax.ShapeDtypeStruct((E_PAD, C), jnp.float32)
                  for _ in range(4)],
        scratch_types=(
            [pltpu.VMEM((CHUNK,), jnp.int32) for _ in range(4)]
            + [pltpu.VMEM((CHUNK, C), jnp.float32) for _ in range(4)]
            + [pltpu.SemaphoreType.DMA for _ in range(4)]
        ),
    )
    def k(x_hbm, idx_hbm, o1, o2, o3, o4,
          i1, i2, i3, i4, r1, r2, r3, r4, s1, s2, s3, s4):
        outs = (o1, o2, o3, o4)
        idxs = (i1, i2, i3, i4)
        rows = (r1, r2, r3, r4)
        sems = (s1, s2, s3, s4)
        wid = lax.axis_index("s") * 2 + lax.axis_index("c")
        base = wid * E_PER_W
        for c in range(N_CHUNKS):
            off = base + c * CHUNK
            for j in range(4):
                pltpu.sync_copy(idx_hbm.at[j, pl.ds(off, CHUNK)], idxs[j])
            cps = []
            for j in range(4):
                cps.append(pltpu.async_copy(x_hbm.at[idxs[j]], rows[j],
                                            sems[j]))
            for j in range(4):
                cps[j].wait()
                pltpu.sync_copy(rows[j], outs[j].at[pl.ds(off, CHUNK)])

    return k(xT, idx4)


def _tc_conv(x_own, g1, g2, g3, g4, W5, b):
    C = x_own.shape[1]
    O = W5.shape[1]
    n_tiles = E_PAD // E_TILE

    def body(x_ref, g1_ref, g2_ref, g3_ref, g4_ref, w_ref, b_ref,
             y_ref, st_ref):
        i = pl.program_id(0)

        @pl.when(i == 0)
        def _():
            st_ref[...] = jnp.zeros_like(st_ref)

        f0 = x_ref[...]
        a1 = g1_ref[...]
        a2 = g2_ref[...]
        a3 = g3_ref[...]
        a4 = g4_ref[...]
        G = jnp.concatenate(
            [f0, a1 + a3, a2 + a4, jnp.abs(a1 - a3), jnp.abs(a2 - a4)],
            axis=1)
        y = jnp.dot(G, w_ref[...], preferred_element_type=jnp.float32)
        y = y + b_ref[...]
        y = jnp.where(y >= 0.0, y, LEAKY * y)
        y_ref[...] = y

        row = i * E_TILE + lax.broadcasted_iota(jnp.int32, (E_TILE, 1), 0)
        ym = jnp.where(row < E_REAL, y, 0.0)
        s = jnp.sum(ym, axis=0, keepdims=True)
        q = jnp.sum(ym * ym, axis=0, keepdims=True)
        st_ref[0:1, :] += s
        st_ref[1:2, :] += q

    y, st = pl.pallas_call(
        body,
        grid=(n_tiles,),
        in_specs=[
            pl.BlockSpec((E_TILE, C), lambda i: (i, 0)),
            pl.BlockSpec((E_TILE, C), lambda i: (i, 0)),
            pl.BlockSpec((E_TILE, C), lambda i: (i, 0)),
            pl.BlockSpec((E_TILE, C), lambda i: (i, 0)),
            pl.BlockSpec((E_TILE, C), lambda i: (i, 0)),
            pl.BlockSpec((5 * C, O), lambda i: (0, 0)),
            pl.BlockSpec((1, O), lambda i: (0, 0)),
        ],
        out_specs=[
            pl.BlockSpec((E_TILE, O), lambda i: (i, 0)),
            pl.BlockSpec((8, O), lambda i: (0, 0)),
        ],
        out_shape=[
            jax.ShapeDtypeStruct((E_PAD, O), jnp.float32),
            jax.ShapeDtypeStruct((8, O), jnp.float32),
        ],
    )(x_own, g1, g2, g3, g4, W5, b)
    return y, st


def _tc_affine(y, scale, shift):
    O = y.shape[1]
    n_tiles = E_PAD // E_TILE

    def body(y_ref, sc_ref, sh_ref, o_ref):
        o_ref[...] = y_ref[...] * sc_ref[...] + sh_ref[...]

    return pl.pallas_call(
        body,
        grid=(n_tiles,),
        in_specs=[
            pl.BlockSpec((E_TILE, O), lambda i: (i, 0)),
            pl.BlockSpec((1, O), lambda i: (0, 0)),
            pl.BlockSpec((1, O), lambda i: (0, 0)),
        ],
        out_specs=pl.BlockSpec((E_TILE, O), lambda i: (i, 0)),
        out_shape=jax.ShapeDtypeStruct((E_PAD, O), jnp.float32),
    )(y, scale, shift)


def _norm_affine(st):
    s = st[0]
    q = st[1]
    m = s / E_REAL
    v = q / E_REAL - m * m
    inv = 1.0 / jnp.sqrt(v + EPS)
    S = (1.0 + 1.0 / jnp.sqrt(v / (v + EPS) + EPS)) * inv
    return S[None, :], (-m * S)[None, :]


def _w5(W):
    return jnp.transpose(W, (2, 1, 0)).reshape(-1, W.shape[0])


def kernel(in_x, gemm_edges, W1_0, b1_0, W2_0, b2_0, W1_1, b1_1, W2_1, b2_1):
    xT = jnp.transpose(in_x[0])
    xT = jnp.pad(xT, ((0, E_PAD - E_REAL), (0, 0)))
    idx = gemm_edges[0].astype(jnp.int32)
    idx4 = jnp.pad(jnp.transpose(idx), ((0, 0), (0, E_PAD - E_REAL)))

    g1, g2, g3, g4 = _sc_gather(xT, idx4)
    W5_0 = _w5(W1_0)
    y0, st0 = _tc_conv(xT, g1, g2, g3, g4, W5_0, b1_0[None, :])
    S0, t0 = _norm_affine(st0)

    W1f = W1_1 * S0[0][None, :, None]
    bf = b1_1 + (W1_1[:, :, 0] + 2.0 * W1_1[:, :, 1]
                 + 2.0 * W1_1[:, :, 2]) @ t0[0]

    h1, h2, h3, h4 = _sc_gather(y0, idx4)
    W5_1 = _w5(W1f)
    y1, st1 = _tc_conv(y0, h1, h2, h3, h4, W5_1, bf[None, :])
    S1, t1 = _norm_affine(st1)

    out = _tc_affine(y1, S1, t1)
    return jnp.transpose(out[:E_REAL])[None]


# speedup vs baseline: 3.7250x; 1.0555x over previous
import functools

import jax
import jax.numpy as jnp
from jax import lax
from jax.experimental import pallas as pl
from jax.experimental.pallas import tpu as pltpu
from jax.experimental.pallas import tpu_sc as plsc

LEAKY = 0.2
EPS = 1e-5
E_REAL = 10000
E_PAD = 10240
N_WORKERS = 32
E_PER_W = E_PAD // N_WORKERS
CHUNK = 32
N_CHUNKS = E_PER_W // CHUNK
E_TILE = 256


def _sc_gather(xT, idxR):
    C = xT.shape[1]
    mesh = plsc.VectorSubcoreMesh(core_axis_name="c", subcore_axis_name="s")

    @functools.partial(
        pl.kernel,
        mesh=mesh,
        out_type=[jax.ShapeDtypeStruct((E_PAD, C), jnp.float32)
                  for _ in range(4)],
        scratch_types=(
            [pltpu.VMEM((N_CHUNKS, 4 * CHUNK), jnp.int32)]
            + [pltpu.VMEM((4 * CHUNK, C), jnp.float32) for _ in range(2)]
            + [pltpu.SemaphoreType.DMA for _ in range(4)]
        ),
    )
    def k(x_hbm, idx_hbm, o1, o2, o3, o4,
          idx_v, b0, b1, g0, g1, w0, w1):
        outs = (o1, o2, o3, o4)
        bufs = (b0, b1)
        gsems = (g0, g1)
        wsems = (w0, w1)
        wid = lax.axis_index("s") * 2 + lax.axis_index("c")
        base = wid * E_PER_W
        pltpu.sync_copy(idx_hbm.at[wid], idx_v)

        gcp = [None] * N_CHUNKS
        wcp = [[] for _ in range(N_CHUNKS)]
        gcp[0] = pltpu.async_copy(x_hbm.at[idx_v.at[0]], bufs[0], gsems[0])
        for c in range(N_CHUNKS):
            p = c & 1
            q = (c + 1) & 1
            if c + 1 < N_CHUNKS:
                if c >= 1:
                    for h in wcp[c - 1]:
                        h.wait()
                gcp[c + 1] = pltpu.async_copy(x_hbm.at[idx_v.at[c + 1]],
                                              bufs[q], gsems[q])
            gcp[c].wait()
            off = base + c * CHUNK
            for j in range(4):
                wcp[c].append(pltpu.async_copy(
                    bufs[p].at[pl.ds(j * CHUNK, CHUNK)],
                    outs[j].at[pl.ds(off, CHUNK)], wsems[p]))
        for h in wcp[N_CHUNKS - 2]:
            h.wait()
        for h in wcp[N_CHUNKS - 1]:
            h.wait()

    return k(xT, idxR)


def _tc_conv(x_own, g1, g2, g3, g4, W5, b):
    C = x_own.shape[1]
    O = W5.shape[1]
    n_tiles = E_PAD // E_TILE

    def body(x_ref, g1_ref, g2_ref, g3_ref, g4_ref, w_ref, b_ref,
             y_ref, st_ref):
        i = pl.program_id(0)

        @pl.when(i == 0)
        def _():
            st_ref[...] = jnp.zeros_like(st_ref)

        f0 = x_ref[...]
        a1 = g1_ref[...]
        a2 = g2_ref[...]
        a3 = g3_ref[...]
        a4 = g4_ref[...]
        G = jnp.concatenate(
            [f0, a1 + a3, a2 + a4, jnp.abs(a1 - a3), jnp.abs(a2 - a4)],
            axis=1)
        y = jnp.dot(G, w_ref[...], preferred_element_type=jnp.float32)
        y = y + b_ref[...]
        y = jnp.where(y >= 0.0, y, LEAKY * y)
        y_ref[...] = y

        row = i * E_TILE + lax.broadcasted_iota(jnp.int32, (E_TILE, 1), 0)
        ym = jnp.where(row < E_REAL, y, 0.0)
        s = jnp.sum(ym, axis=0, keepdims=True)
        q = jnp.sum(ym * ym, axis=0, keepdims=True)
        st_ref[0:1, :] += s
        st_ref[1:2, :] += q

    y, st = pl.pallas_call(
        body,
        grid=(n_tiles,),
        in_specs=[
            pl.BlockSpec((E_TILE, C), lambda i: (i, 0)),
            pl.BlockSpec((E_TILE, C), lambda i: (i, 0)),
            pl.BlockSpec((E_TILE, C), lambda i: (i, 0)),
            pl.BlockSpec((E_TILE, C), lambda i: (i, 0)),
            pl.BlockSpec((E_TILE, C), lambda i: (i, 0)),
            pl.BlockSpec((5 * C, O), lambda i: (0, 0)),
            pl.BlockSpec((1, O), lambda i: (0, 0)),
        ],
        out_specs=[
            pl.BlockSpec((E_TILE, O), lambda i: (i, 0)),
            pl.BlockSpec((8, O), lambda i: (0, 0)),
        ],
        out_shape=[
            jax.ShapeDtypeStruct((E_PAD, O), jnp.float32),
            jax.ShapeDtypeStruct((8, O), jnp.float32),
        ],
    )(x_own, g1, g2, g3, g4, W5, b)
    return y, st


def _tc_affine(y, scale, shift):
    O = y.shape[1]
    n_tiles = E_PAD // E_TILE

    def body(y_ref, sc_ref, sh_ref, o_ref):
        o_ref[...] = y_ref[...] * sc_ref[...] + sh_ref[...]

    return pl.pallas_call(
        body,
        grid=(n_tiles,),
        in_specs=[
            pl.BlockSpec((E_TILE, O), lambda i: (i, 0)),
            pl.BlockSpec((1, O), lambda i: (0, 0)),
            pl.BlockSpec((1, O), lambda i: (0, 0)),
        ],
        out_specs=pl.BlockSpec((E_TILE, O), lambda i: (i, 0)),
        out_shape=jax.ShapeDtypeStruct((E_PAD, O), jnp.float32),
    )(y, scale, shift)


def _norm_affine(st):
    s = st[0]
    q = st[1]
    m = s / E_REAL
    v = q / E_REAL - m * m
    inv = 1.0 / jnp.sqrt(v + EPS)
    S = (1.0 + 1.0 / jnp.sqrt(v / (v + EPS) + EPS)) * inv
    return S[None, :], (-m * S)[None, :]


def _w5(W):
    return jnp.transpose(W, (2, 1, 0)).reshape(-1, W.shape[0])


def kernel(in_x, gemm_edges, W1_0, b1_0, W2_0, b2_0, W1_1, b1_1, W2_1, b2_1):
    xT = jnp.transpose(in_x[0])
    xT = jnp.pad(xT, ((0, E_PAD - E_REAL), (0, 0)))
    idx = gemm_edges[0].astype(jnp.int32)
    idx4 = jnp.pad(jnp.transpose(idx), ((0, 0), (0, E_PAD - E_REAL)))
    idxR = jnp.transpose(
        idx4.reshape(4, N_WORKERS, N_CHUNKS, CHUNK),
        (1, 2, 0, 3)).reshape(N_WORKERS, N_CHUNKS, 4 * CHUNK)

    g1, g2, g3, g4 = _sc_gather(xT, idxR)
    W5_0 = _w5(W1_0)
    y0, st0 = _tc_conv(xT, g1, g2, g3, g4, W5_0, b1_0[None, :])
    S0, t0 = _norm_affine(st0)

    W1f = W1_1 * S0[0][None, :, None]
    bf = b1_1 + (W1_1[:, :, 0] + 2.0 * W1_1[:, :, 1]
                 + 2.0 * W1_1[:, :, 2]) @ t0[0]

    h1, h2, h3, h4 = _sc_gather(y0, idxR)
    W5_1 = _w5(W1f)
    y1, st1 = _tc_conv(y0, h1, h2, h3, h4, W5_1, bf[None, :])
    S1, t1 = _norm_affine(st1)

    out = _tc_affine(y1, S1, t1)
    return jnp.transpose(out[:E_REAL])[None]


# speedup vs baseline: 4.0528x; 1.0880x over previous
import functools

import jax
import jax.numpy as jnp
from jax import lax
from jax.experimental import pallas as pl
from jax.experimental.pallas import tpu as pltpu
from jax.experimental.pallas import tpu_sc as plsc

LEAKY = 0.2
EPS = 1e-5
E_REAL = 10000
E_PAD = 10240
N_WORKERS = 32
E_PER_W = E_PAD // N_WORKERS
CHUNK = 32
N_CHUNKS = E_PER_W // CHUNK
E_TILE = 256


def _sc_gather(xT, idxR):
    C = xT.shape[1]
    n_gchunks = E_PAD // CHUNK
    mesh = plsc.VectorSubcoreMesh(core_axis_name="c", subcore_axis_name="s")

    @functools.partial(
        pl.kernel,
        mesh=mesh,
        out_type=jax.ShapeDtypeStruct((n_gchunks, 4 * CHUNK, C),
                                      jnp.float32),
        scratch_types=(
            [pltpu.VMEM((N_CHUNKS, 4 * CHUNK), jnp.int32)]
            + [pltpu.VMEM((4 * CHUNK, C), jnp.float32) for _ in range(2)]
            + [pltpu.SemaphoreType.DMA for _ in range(4)]
        ),
    )
    def k(x_hbm, idx_hbm, out,
          idx_v, b0, b1, g0, g1, w0, w1):
        bufs = (b0, b1)
        gsems = (g0, g1)
        wsems = (w0, w1)
        wid = lax.axis_index("s") * 2 + lax.axis_index("c")
        gbase = wid * N_CHUNKS
        pltpu.sync_copy(idx_hbm.at[wid], idx_v)

        gcp = [None] * N_CHUNKS
        wcp = [None] * N_CHUNKS
        gcp[0] = pltpu.async_copy(x_hbm.at[idx_v.at[0]], bufs[0], gsems[0])
        for c in range(N_CHUNKS):
            p = c & 1
            q = (c + 1) & 1
            if c + 1 < N_CHUNKS:
                if c >= 1:
                    wcp[c - 1].wait()
                gcp[c + 1] = pltpu.async_copy(x_hbm.at[idx_v.at[c + 1]],
                                              bufs[q], gsems[q])
            gcp[c].wait()
            wcp[c] = pltpu.async_copy(bufs[p], out.at[gbase + c], wsems[p])
        wcp[N_CHUNKS - 2].wait()
        wcp[N_CHUNKS - 1].wait()

    return k(xT, idxR)


def _tc_conv(x_own, gath, W5, b):
    C = x_own.shape[1]
    O = W5.shape[1]
    n_tiles = E_PAD // E_TILE
    cpt = E_TILE // CHUNK

    def body(x_ref, g1_ref, g2_ref, g3_ref, g4_ref, w_ref, b_ref,
             y_ref, st_ref):
        i = pl.program_id(0)

        @pl.when(i == 0)
        def _():
            st_ref[...] = jnp.zeros_like(st_ref)

        f0 = x_ref[...]
        a1 = g1_ref[...].reshape(E_TILE, C)
        a2 = g2_ref[...].reshape(E_TILE, C)
        a3 = g3_ref[...].reshape(E_TILE, C)
        a4 = g4_ref[...].reshape(E_TILE, C)
        G = jnp.concatenate(
            [f0, a1 + a3, a2 + a4, jnp.abs(a1 - a3), jnp.abs(a2 - a4)],
            axis=1)
        y = jnp.dot(G, w_ref[...], preferred_element_type=jnp.float32)
        y = y + b_ref[...]
        y = jnp.where(y >= 0.0, y, LEAKY * y)
        y_ref[...] = y

        row = i * E_TILE + lax.broadcasted_iota(jnp.int32, (E_TILE, 1), 0)
        ym = jnp.where(row < E_REAL, y, 0.0)
        s = jnp.sum(ym, axis=0, keepdims=True)
        q = jnp.sum(ym * ym, axis=0, keepdims=True)
        st_ref[0:1, :] += s
        st_ref[1:2, :] += q

    y, st = pl.pallas_call(
        body,
        grid=(n_tiles,),
        in_specs=[
            pl.BlockSpec((E_TILE, C), lambda i: (i, 0)),
            pl.BlockSpec((cpt, CHUNK, C), lambda i: (i, 0, 0)),
            pl.BlockSpec((cpt, CHUNK, C), lambda i: (i, 1, 0)),
            pl.BlockSpec((cpt, CHUNK, C), lambda i: (i, 2, 0)),
            pl.BlockSpec((cpt, CHUNK, C), lambda i: (i, 3, 0)),
            pl.BlockSpec((5 * C, O), lambda i: (0, 0)),
            pl.BlockSpec((1, O), lambda i: (0, 0)),
        ],
        out_specs=[
            pl.BlockSpec((E_TILE, O), lambda i: (i, 0)),
            pl.BlockSpec((8, O), lambda i: (0, 0)),
        ],
        out_shape=[
            jax.ShapeDtypeStruct((E_PAD, O), jnp.float32),
            jax.ShapeDtypeStruct((8, O), jnp.float32),
        ],
    )(x_own, gath, gath, gath, gath, W5, b)
    return y, st


def _tc_affine(y, scale, shift):
    O = y.shape[1]
    n_tiles = E_PAD // E_TILE

    def body(y_ref, sc_ref, sh_ref, o_ref):
        o_ref[...] = y_ref[...] * sc_ref[...] + sh_ref[...]

    return pl.pallas_call(
        body,
        grid=(n_tiles,),
        in_specs=[
            pl.BlockSpec((E_TILE, O), lambda i: (i, 0)),
            pl.BlockSpec((1, O), lambda i: (0, 0)),
            pl.BlockSpec((1, O), lambda i: (0, 0)),
        ],
        out_specs=pl.BlockSpec((E_TILE, O), lambda i: (i, 0)),
        out_shape=jax.ShapeDtypeStruct((E_PAD, O), jnp.float32),
    )(y, scale, shift)


def _norm_affine(st):
    s = st[0]
    q = st[1]
    m = s / E_REAL
    v = q / E_REAL - m * m
    inv = 1.0 / jnp.sqrt(v + EPS)
    S = (1.0 + 1.0 / jnp.sqrt(v / (v + EPS) + EPS)) * inv
    return S[None, :], (-m * S)[None, :]


def _w5(W):
    return jnp.transpose(W, (2, 1, 0)).reshape(-1, W.shape[0])


def kernel(in_x, gemm_edges, W1_0, b1_0, W2_0, b2_0, W1_1, b1_1, W2_1, b2_1):
    xT = jnp.transpose(in_x[0])
    xT = jnp.pad(xT, ((0, E_PAD - E_REAL), (0, 0)))
    idx = gemm_edges[0].astype(jnp.int32)
    idx4 = jnp.pad(jnp.transpose(idx), ((0, 0), (0, E_PAD - E_REAL)))
    idxR = jnp.transpose(
        idx4.reshape(4, N_WORKERS, N_CHUNKS, CHUNK),
        (1, 2, 0, 3)).reshape(N_WORKERS, N_CHUNKS, 4 * CHUNK)

    g0 = _sc_gather(xT, idxR)
    W5_0 = _w5(W1_0)
    y0, st0 = _tc_conv(xT, g0, W5_0, b1_0[None, :])
    S0, t0 = _norm_affine(st0)

    W1f = W1_1 * S0[0][None, :, None]
    bf = b1_1 + (W1_1[:, :, 0] + 2.0 * W1_1[:, :, 1]
                 + 2.0 * W1_1[:, :, 2]) @ t0[0]

    g1 = _sc_gather(y0, idxR)
    W5_1 = _w5(W1f)
    y1, st1 = _tc_conv(y0, g1, W5_1, bf[None, :])
    S1, t1 = _norm_affine(st1)

    out = _tc_affine(y1, S1, t1)
    return jnp.transpose(out[:E_REAL])[None]
